# baseline (speedup 1.0000x reference)
import functools

import jax
import jax.numpy as jnp
from jax import lax
from jax.experimental import pallas as pl
from jax.experimental.pallas import tpu as pltpu
from jax.experimental.pallas import tpu_sc as plsc

VOCAB = 1000
NBATCH = 4096
T = 20
NC = 2
NS = 16
NW = NC * NS
L = 16
VT = VOCAB // 8
VT_SPLITS = (31, 31, 31, 32)
VT_BASES = (0, 31, 62, 93)
BT = NBATCH // 128


def _one_hot_body(xt_hbm, out_hbm, idx_v, buf0, buf1, buf2, buf3,
                  sem0, sem1, sem2, sem3):
    bufs = (buf0, buf1, buf2, buf3)
    sems = (sem0, sem1, sem2, sem3)
    cid = lax.axis_index("c")
    sid = lax.axis_index("s")
    wid = sid * NC + cid

    pltpu.sync_copy(xt_hbm.at[pl.ds(0, T), pl.ds(wid * 128, 128)], idx_v)

    zeros = jnp.zeros((L,), jnp.float32)
    ones = jnp.full((L,), 1.0, jnp.float32)
    lane = lax.iota(jnp.int32, L)
    zlane = lane * 0

    def zero_buf(k):
        def body(i, carry):
            vt = i // 8
            vi = i % 8
            for c0 in range(0, 128, L):
                bufs[k][0, vt, 0, vi, pl.ds(c0, L)] = zeros
            return carry
        lax.fori_loop(0, VT_SPLITS[k] * 8, body, 0)

    def scatter(k, t, val):
        lo, n = VT_BASES[k], VT_SPLITS[k]
        for g in range(128 // L):
            col = idx_v[t, pl.ds(g * L, L)]
            blane = lane + g * L
            vt = col // 8
            vtk = jnp.clip(vt - lo, 0, n - 1)
            mask = jnp.logical_and(vt >= lo, vt < lo + n)
            plsc.store_scatter(
                bufs[k], [zlane, vtk, zlane, col % 8, blane], val,
                mask=mask)

    def dst(k, t):
        return out_hbm.at[pl.ds(t, 1), pl.ds(VT_BASES[k], VT_SPLITS[k]),
                          pl.ds(wid, 1), pl.ds(0, 8), pl.ds(0, 128)]

    def start_dma(k, t):
        pltpu.make_async_copy(bufs[k], dst(k, t), sems[k]).start()

    def wait_dma(k):
        pltpu.make_async_copy(bufs[k], dst(k, 0), sems[k]).wait()

    for k in range(4):
        zero_buf(k)
        scatter(k, 0, ones)
        start_dma(k, 0)

    def loop_body(t, carry):
        for k in range(4):
            wait_dma(k)
            scatter(k, t - 1, zeros)
            scatter(k, t, ones)
            start_dma(k, t)
        return carry

    lax.fori_loop(1, T, loop_body, 0)
    for k in range(4):
        wait_dma(k)


_one_hot_sc = functools.partial(
    pl.kernel,
    out_type=jax.ShapeDtypeStruct((T, VT, BT, 8, 128), jnp.float32),
    mesh=plsc.VectorSubcoreMesh(
        core_axis_name="c", subcore_axis_name="s",
        num_cores=NC, num_subcores=NS),
    compiler_params=pltpu.CompilerParams(needs_layout_passes=False),
    scratch_types=(
        [pltpu.VMEM((T, 128), jnp.int32)]
        + [pltpu.VMEM((1, n, 1, 8, 128), jnp.float32) for n in VT_SPLITS]
        + [pltpu.SemaphoreType.DMA] * 4
    ),
)(_one_hot_body)


@jax.jit
def kernel(x, table):
    del table
    xt = jnp.transpose(x)
    out5 = _one_hot_sc(xt)
    return out5.transpose(2, 4, 0, 1, 3).reshape(NBATCH, T, VOCAB)


# speedup vs baseline: 1.0016x; 1.0016x over previous
import functools

import jax
import jax.numpy as jnp
from jax import lax
from jax.experimental import pallas as pl
from jax.experimental.pallas import tpu as pltpu
from jax.experimental.pallas import tpu_sc as plsc

VOCAB = 1000
NBATCH = 4096
T = 20
NC = 2
NS = 16
NW = NC * NS
L = 16
VT = VOCAB // 8
VTA = 63
VTB = VT - VTA
BT = NBATCH // 128


def _one_hot_body(xt_hbm, out_hbm, idx_v, buf_a, buf_b, sem_a, sem_b):
    cid = lax.axis_index("c")
    sid = lax.axis_index("s")
    wid = sid * NC + cid

    pltpu.sync_copy(xt_hbm.at[pl.ds(0, T), pl.ds(wid * 128, 128)], idx_v)

    zeros = jnp.zeros((L,), jnp.float32)
    ones = jnp.full((L,), 1.0, jnp.float32)
    lane = lax.iota(jnp.int32, L)
    zlane = lane * 0

    def zero_buf(buf, nvt):
        def body(i, carry):
            vt = i // 8
            vi = i % 8
            for c0 in range(0, 128, L):
                buf[0, vt, 0, vi, pl.ds(c0, L)] = zeros
            return carry
        lax.fori_loop(0, nvt * 8, body, 0)

    def scatter_a(t, val):
        for g in range(128 // L):
            col = idx_v[t, pl.ds(g * L, L)]
            blane = lane + g * L
            vt = col // 8
            plsc.store_scatter(
                buf_a, [zlane, jnp.minimum(vt, VTA - 1), zlane, col % 8,
                        blane], val, mask=vt < VTA)

    def scatter_b(t, val):
        for g in range(128 // L):
            col = idx_v[t, pl.ds(g * L, L)]
            blane = lane + g * L
            vt = col // 8
            plsc.store_scatter(
                buf_b, [zlane, jnp.maximum(vt - VTA, 0), zlane, col % 8,
                        blane], val, mask=vt >= VTA)

    def dst_a(t):
        return out_hbm.at[pl.ds(t, 1), pl.ds(0, VTA), pl.ds(wid, 1),
                          pl.ds(0, 8), pl.ds(0, 128)]

    def dst_b(t):
        return out_hbm.at[pl.ds(t, 1), pl.ds(VTA, VTB), pl.ds(wid, 1),
                          pl.ds(0, 8), pl.ds(0, 128)]

    def start_a(t):
        pltpu.make_async_copy(buf_a, dst_a(t), sem_a).start()

    def start_b(t):
        pltpu.make_async_copy(buf_b, dst_b(t), sem_b).start()

    def wait_a():
        pltpu.make_async_copy(buf_a, dst_a(0), sem_a).wait()

    def wait_b():
        pltpu.make_async_copy(buf_b, dst_b(0), sem_b).wait()

    zero_buf(buf_a, VTA)
    scatter_a(0, ones)
    start_a(0)
    zero_buf(buf_b, VTB)
    scatter_b(0, ones)
    start_b(0)

    def loop_body(t, carry):
        wait_a()
        scatter_a(t - 1, zeros)
        scatter_a(t, ones)
        start_a(t)
        wait_b()
        scatter_b(t - 1, zeros)
        scatter_b(t, ones)
        start_b(t)
        return carry

    lax.fori_loop(1, T, loop_body, 0)
    wait_a()
    wait_b()


_one_hot_sc = functools.partial(
    pl.kernel,
    out_type=jax.ShapeDtypeStruct((T, VT, BT, 8, 128), jnp.float32),
    mesh=plsc.VectorSubcoreMesh(
        core_axis_name="c", subcore_axis_name="s",
        num_cores=NC, num_subcores=NS),
    compiler_params=pltpu.CompilerParams(needs_layout_passes=False),
    scratch_types=[
        pltpu.VMEM((T, 128), jnp.int32),
        pltpu.VMEM((1, VTA, 1, 8, 128), jnp.float32),
        pltpu.VMEM((1, VTB, 1, 8, 128), jnp.float32),
        pltpu.SemaphoreType.DMA,
        pltpu.SemaphoreType.DMA,
    ],
)(_one_hot_body)


@jax.jit
def kernel(x, table):
    del table
    xt = jnp.transpose(x)
    out5 = _one_hot_sc(xt)
    return out5.transpose(2, 4, 0, 1, 3).reshape(NBATCH, T, VOCAB)
